# baseline (speedup 1.0000x reference)
import functools

import jax
import jax.numpy as jnp
from jax import lax
from jax.experimental import pallas as pl
from jax.experimental.pallas import tpu as pltpu
from jax.experimental.pallas import tpu_sc as plsc

_NC = 2
_NS = 16
_NW = _NC * _NS

_R = 4
_L = 500000
_SLAB = 4096
_NSLAB = _L // _SLAB
_TAIL = _L - _NSLAB * _SLAB
_ROUNDS = (_NSLAB + _NW - 1) // _NW
_UNROLL = 1


def _dyn_gather(v, idx):
    return lax.gather(
        v,
        idx[:, None],
        dimension_numbers=lax.GatherDimensionNumbers(
            offset_dims=(), collapsed_slice_dims=(0,), start_index_map=(0,)
        ),
        slice_sizes=(1,),
        mode=lax.GatherScatterMode.PROMISE_IN_BOUNDS,
    )


def _sc_body(
    x_hbm, v_hbm, o_hbm, vbuf, xb0, xb1, ob0, ob1, xtail, otail, is0, is1, os0, os1
):
    cid = lax.axis_index("c")
    sid = lax.axis_index("s")
    w = sid * _NC + cid
    xbufs = (xb0, xb1)
    obufs = (ob0, ob1)
    isems = (is0, is1)
    osems = (os0, os1)

    pltpu.sync_copy(v_hbm, vbuf)

    iota = lax.iota(jnp.int32, 16)
    shift = jnp.minimum(iota + 1, 15)
    t = iota + 1
    pow_d = jnp.where(t >= 8, 8, jnp.where(t >= 4, 4, jnp.where(t >= 2, 2, 1)))
    pow_3md = jnp.where(t >= 8, 1, jnp.where(t >= 4, 2, jnp.where(t >= 2, 4, 8)))
    hperm = jnp.minimum((2 * (t - pow_d) + 1) * pow_3md - 1, 15)
    zeros = iota * 0

    svs = []
    hms = []
    roots = []
    for r0 in range(_R):
        v = vbuf[r0, :]
        sv = v
        for lk in (1, 2, 3, 4):
            for lj in range(lk - 1, -1, -1):
                partner = jnp.bitwise_xor(iota, 1 << lj)
                pv = _dyn_gather(sv, partner)
                disagree = jnp.bitwise_and(
                    jnp.bitwise_xor(
                        lax.shift_right_logical(iota, lj),
                        lax.shift_right_logical(iota, lk),
                    ),
                    1,
                )
                sv = jnp.where(
                    disagree == 0, jnp.minimum(sv, pv), jnp.maximum(sv, pv)
                )
        mid = (sv + _dyn_gather(sv, shift)) * 0.5
        hm = _dyn_gather(mid, hperm)
        svs.append(sv)
        hms.append(hm)
        roots.append(_dyn_gather(hm, zeros))

    def search(x, r0):
        idx = jnp.where(x > roots[r0], 2, 1)
        for _lvl in range(3):
            g = _dyn_gather(hms[r0], idx)
            idx = idx + idx + jnp.where(x > g, 2, 1)
        return _dyn_gather(svs[r0], idx - 15)

    def slab_src(k):
        return x_hbm.at[:, pl.ds((w + _NW * k) * _SLAB, _SLAB)]

    def slab_dst(k):
        return o_hbm.at[:, pl.ds((w + _NW * k) * _SLAB, _SLAB)]

    def valid(k):
        return w + _NW * k < _NSLAB

    @pl.when(valid(0))
    def _():
        pltpu.async_copy(slab_src(0), xbufs[0], isems[0])

    for k in range(_ROUNDS):
        b = k % 2

        @pl.when(valid(k))
        def _(k=k, b=b):
            pltpu.make_async_copy(slab_src(k), xbufs[b], isems[b]).wait()
            if k + 1 < _ROUNDS:

                @pl.when(valid(k + 1))
                def _():
                    pltpu.async_copy(slab_src(k + 1), xbufs[1 - b], isems[1 - b])

            if k >= 2:
                pltpu.make_async_copy(obufs[b], slab_dst(k - 2), osems[b]).wait()

            for r0 in range(_R):

                @plsc.parallel_loop(0, _SLAB // 16, unroll=_UNROLL)
                def _(i, r0=r0):
                    xv = xbufs[b][r0, pl.ds(i * 16, 16)]
                    obufs[b][r0, pl.ds(i * 16, 16)] = search(xv, r0)

            pltpu.async_copy(obufs[b], slab_dst(k), osems[b])

    for k in range(_ROUNDS):
        not_covered = (
            valid(k)
            if k + 2 >= _ROUNDS
            else jnp.logical_and(valid(k), jnp.logical_not(valid(k + 2)))
        )

        @pl.when(not_covered)
        def _(k=k):
            pltpu.make_async_copy(obufs[k % 2], slab_dst(k), osems[k % 2]).wait()

    @pl.when(w == _NW - 1)
    def _():
        base = _NSLAB * _SLAB
        pltpu.sync_copy(x_hbm.at[:, pl.ds(base, _TAIL)], xtail)
        for r0 in range(_R):
            for i in range(_TAIL // 16):
                xv = xtail[r0, pl.ds(i * 16, 16)]
                otail[r0, pl.ds(i * 16, 16)] = search(xv, r0)
        pltpu.sync_copy(otail, o_hbm.at[:, pl.ds(base, _TAIL)])


def kernel(x, values):
    sck = functools.partial(
        pl.kernel,
        mesh=plsc.VectorSubcoreMesh(core_axis_name="c", subcore_axis_name="s"),
        out_type=jax.ShapeDtypeStruct((_R, _L), jnp.float32),
        scratch_types=[
            pltpu.VMEM((_R, 16), jnp.float32),
            pltpu.VMEM((_R, _SLAB), jnp.float32),
            pltpu.VMEM((_R, _SLAB), jnp.float32),
            pltpu.VMEM((_R, _SLAB), jnp.float32),
            pltpu.VMEM((_R, _SLAB), jnp.float32),
            pltpu.VMEM((_R, _TAIL), jnp.float32),
            pltpu.VMEM((_R, _TAIL), jnp.float32),
            pltpu.SemaphoreType.DMA,
            pltpu.SemaphoreType.DMA,
            pltpu.SemaphoreType.DMA,
            pltpu.SemaphoreType.DMA,
        ],
    )(_sc_body)
    return sck(x, values)


# speedup vs baseline: 1.0963x; 1.0963x over previous
import functools

import jax
import jax.numpy as jnp
from jax import lax
from jax.experimental import pallas as pl
from jax.experimental.pallas import tpu as pltpu
from jax.experimental.pallas import tpu_sc as plsc

_NC = 2
_NS = 16
_NW = _NC * _NS

_R = 4
_L = 500000
_SLAB = 4096
_NSLAB = _L // _SLAB
_TAIL = _L - _NSLAB * _SLAB
_ROUNDS = (_NSLAB + _NW - 1) // _NW
_UNROLL = 2


def _dyn_gather(v, idx):
    return lax.gather(
        v,
        idx[:, None],
        dimension_numbers=lax.GatherDimensionNumbers(
            offset_dims=(), collapsed_slice_dims=(0,), start_index_map=(0,)
        ),
        slice_sizes=(1,),
        mode=lax.GatherScatterMode.PROMISE_IN_BOUNDS,
    )


def _sc_body(
    x_hbm, v_hbm, o_hbm, vbuf, xb0, xb1, ob0, ob1, xtail, otail, is0, is1, os0, os1
):
    cid = lax.axis_index("c")
    sid = lax.axis_index("s")
    w = sid * _NC + cid
    xbufs = (xb0, xb1)
    obufs = (ob0, ob1)
    isems = (is0, is1)
    osems = (os0, os1)

    pltpu.sync_copy(v_hbm, vbuf)

    iota = lax.iota(jnp.int32, 16)
    shift = jnp.minimum(iota + 1, 15)
    t = iota + 1
    pow_d = jnp.where(t >= 8, 8, jnp.where(t >= 4, 4, jnp.where(t >= 2, 2, 1)))
    pow_3md = jnp.where(t >= 8, 1, jnp.where(t >= 4, 2, jnp.where(t >= 2, 4, 8)))
    hperm = jnp.minimum((2 * (t - pow_d) + 1) * pow_3md - 1, 15)
    zeros = iota * 0

    svs = []
    hms = []
    roots = []
    for r0 in range(_R):
        v = vbuf[r0, :]
        sv = v
        for lk in (1, 2, 3, 4):
            for lj in range(lk - 1, -1, -1):
                partner = jnp.bitwise_xor(iota, 1 << lj)
                pv = _dyn_gather(sv, partner)
                disagree = jnp.bitwise_and(
                    jnp.bitwise_xor(
                        lax.shift_right_logical(iota, lj),
                        lax.shift_right_logical(iota, lk),
                    ),
                    1,
                )
                sv = jnp.where(
                    disagree == 0, jnp.minimum(sv, pv), jnp.maximum(sv, pv)
                )
        mid = (sv + _dyn_gather(sv, shift)) * 0.5
        hm = _dyn_gather(mid, hperm)
        svs.append(sv)
        hms.append(hm)
        roots.append(_dyn_gather(hm, zeros))

    def search(x, r0):
        idx = jnp.where(x > roots[r0], 2, 1)
        for _lvl in range(3):
            g = _dyn_gather(hms[r0], idx)
            idx = idx + idx + jnp.where(x > g, 2, 1)
        return _dyn_gather(svs[r0], idx - 15)

    def slab_src(k):
        return x_hbm.at[:, pl.ds((w + _NW * k) * _SLAB, _SLAB)]

    def slab_dst(k):
        return o_hbm.at[:, pl.ds((w + _NW * k) * _SLAB, _SLAB)]

    def valid(k):
        return w + _NW * k < _NSLAB

    @pl.when(valid(0))
    def _():
        pltpu.async_copy(slab_src(0), xbufs[0], isems[0])

    for k in range(_ROUNDS):
        b = k % 2

        @pl.when(valid(k))
        def _(k=k, b=b):
            pltpu.make_async_copy(slab_src(k), xbufs[b], isems[b]).wait()
            if k + 1 < _ROUNDS:

                @pl.when(valid(k + 1))
                def _():
                    pltpu.async_copy(slab_src(k + 1), xbufs[1 - b], isems[1 - b])

            if k >= 2:
                pltpu.make_async_copy(obufs[b], slab_dst(k - 2), osems[b]).wait()

            for r0 in range(_R):

                @plsc.parallel_loop(0, _SLAB // 16, unroll=_UNROLL)
                def _(i, r0=r0):
                    xv = xbufs[b][r0, pl.ds(i * 16, 16)]
                    obufs[b][r0, pl.ds(i * 16, 16)] = search(xv, r0)

            pltpu.async_copy(obufs[b], slab_dst(k), osems[b])

    for k in range(_ROUNDS):
        not_covered = (
            valid(k)
            if k + 2 >= _ROUNDS
            else jnp.logical_and(valid(k), jnp.logical_not(valid(k + 2)))
        )

        @pl.when(not_covered)
        def _(k=k):
            pltpu.make_async_copy(obufs[k % 2], slab_dst(k), osems[k % 2]).wait()

    @pl.when(w == _NW - 1)
    def _():
        base = _NSLAB * _SLAB
        pltpu.sync_copy(x_hbm.at[:, pl.ds(base, _TAIL)], xtail)
        for r0 in range(_R):
            for i in range(_TAIL // 16):
                xv = xtail[r0, pl.ds(i * 16, 16)]
                otail[r0, pl.ds(i * 16, 16)] = search(xv, r0)
        pltpu.sync_copy(otail, o_hbm.at[:, pl.ds(base, _TAIL)])


def kernel(x, values):
    sck = functools.partial(
        pl.kernel,
        mesh=plsc.VectorSubcoreMesh(core_axis_name="c", subcore_axis_name="s"),
        out_type=jax.ShapeDtypeStruct((_R, _L), jnp.float32),
        scratch_types=[
            pltpu.VMEM((_R, 16), jnp.float32),
            pltpu.VMEM((_R, _SLAB), jnp.float32),
            pltpu.VMEM((_R, _SLAB), jnp.float32),
            pltpu.VMEM((_R, _SLAB), jnp.float32),
            pltpu.VMEM((_R, _SLAB), jnp.float32),
            pltpu.VMEM((_R, _TAIL), jnp.float32),
            pltpu.VMEM((_R, _TAIL), jnp.float32),
            pltpu.SemaphoreType.DMA,
            pltpu.SemaphoreType.DMA,
            pltpu.SemaphoreType.DMA,
            pltpu.SemaphoreType.DMA,
        ],
    )(_sc_body)
    return sck(x, values)
